# Initial kernel scaffold; baseline (speedup 1.0000x reference)
import jax
import jax.numpy as jnp
from jax.experimental import pallas as pl


def kernel(streams_data, segment_ids, pe_global, q_cells, W_embed, b_embed, Wq, Wk, Wv, Wo, W1, b1, W2, b2, Wq2, Wk2, Wv2, Wo2, Wqg, Wkg, Wvg, Wog, Wg1, bg1, Wg2, bg2):
    raise NotImplementedError("write your pallas kernel here")


# speedup vs baseline: 1.3196x; 1.3196x over previous
import functools

import jax
import jax.numpy as jnp
from jax import lax
from jax.experimental import pallas as pl
from jax.experimental.pallas import tpu as pltpu

C = 768; T = 24576; SRC = 128; D = 256; L = 96; H = 4; DH = 64; Q = 4; FF = 1024
S = C * Q
CB = 4
N = CB * L
NSEG = 12
TB = T // NSEG
QB = 768
SCALE = 0.125
NEG = -1e9


def _seg_kernel(seg_ref, starts_ref, lens_ref, acc_lt, acc_eq):
    i = pl.program_id(0)

    @pl.when(i == 0)
    def _init():
        acc_lt[...] = jnp.zeros_like(acc_lt)
        acc_eq[...] = jnp.zeros_like(acc_eq)

    seg = seg_ref[0]
    cells = lax.broadcasted_iota(jnp.int32, (1, C), 1)
    lt = (seg < cells).astype(jnp.float32)
    eq = (seg == cells).astype(jnp.float32)
    acc_lt[...] += jnp.sum(lt, axis=0, keepdims=True)
    acc_eq[...] += jnp.sum(eq, axis=0, keepdims=True)

    @pl.when(i == NSEG - 1)
    def _fin():
        starts_ref[...] = acc_lt[...].astype(jnp.int32)
        lens_ref[...] = acc_eq[...].astype(jnp.int32)


def _masked_attn(qh, kh, vh, allow, allowf):
    s = lax.dot_general(qh, kh, (((1,), (1,)), ((), ()))) * SCALE
    s = jnp.where(allow, s, NEG)
    m = jnp.max(s, axis=1, keepdims=True)
    p = jnp.exp(s - m) * allowf
    den = jnp.maximum(jnp.sum(p, axis=1, keepdims=True), 1e-30)
    return (p @ vh) / den


def _local_kernel(starts, lens, streams, qc, pe, we, be, wq, wk, wv, wo,
                  w1, b1, w2, b2, wq2, wk2, wv2, wo2, out, xbuf, sem):
    pid = pl.program_id(0)
    c0 = pid * CB
    shifts, clens, copies = [], [], []
    for j in range(CB):
        st = starts[c0 + j]
        cl = jnp.minimum(lens[c0 + j], L)
        s0 = jnp.minimum(st, T - L)
        shifts.append(st - s0)
        clens.append(cl)
        cp = pltpu.make_async_copy(streams.at[pl.ds(s0, L), :], xbuf.at[j],
                                   sem.at[j])
        cp.start()
        copies.append(cp)

    rcol = lax.broadcasted_iota(jnp.int32, (N, 1), 0)
    rrow = lax.broadcasted_iota(jnp.int32, (1, N), 1)

    def valid_of(r):
        cidx = r // L
        rl = r % L
        sh = jnp.zeros_like(r)
        cl = jnp.zeros_like(r)
        for j in range(CB):
            sh = jnp.where(cidx == j, shifts[j], sh)
            cl = jnp.where(cidx == j, clens[j], cl)
        return (rl >= sh) & (rl < sh + cl)

    vcol = valid_of(rcol)
    vrow = valid_of(rrow)
    maskf = vcol.astype(jnp.float32)

    for cp in copies:
        cp.wait()

    x = xbuf[...].reshape(N, SRC) @ we[...] + be[...]
    x = x * maskf

    q = x @ wq[...]; k = x @ wk[...]; v = x @ wv[...]
    ri = lax.broadcasted_iota(jnp.int32, (N, N), 0) // L
    ci = lax.broadcasted_iota(jnp.int32, (N, N), 1) // L
    allow = (ri == ci) & vrow
    allowf = allow.astype(jnp.float32)
    os_ = [_masked_attn(q[:, DH*h:DH*(h+1)], k[:, DH*h:DH*(h+1)],
                        v[:, DH*h:DH*(h+1)], allow, allowf) for h in range(H)]
    o = jnp.concatenate(os_, axis=1) @ wo[...]
    h1 = x + o * maskf
    h2 = h1 + (jax.nn.gelu(h1 @ w1[...] + b1[...]) @ w2[...] + b2[...]) * maskf

    k2 = h2 @ wk2[...]; v2 = h2 @ wv2[...]
    qpe = (qc[...] + pe[...]).reshape(CB * Q, D)
    qg = qpe @ wq2[...]
    ri2 = lax.broadcasted_iota(jnp.int32, (CB * Q, N), 0) // Q
    ci2 = lax.broadcasted_iota(jnp.int32, (CB * Q, N), 1) // L
    allow2 = (ri2 == ci2) & vrow
    allow2f = allow2.astype(jnp.float32)
    gs = [_masked_attn(qg[:, DH*h:DH*(h+1)], k2[:, DH*h:DH*(h+1)],
                       v2[:, DH*h:DH*(h+1)], allow2, allow2f) for h in range(H)]
    g = jnp.concatenate(gs, axis=1) @ wo2[...]
    out[...] = (qpe + pe[...].reshape(CB * Q, D) + g).reshape(CB, Q, D)


def _kv_kernel(tg_ref, wkg, wvg, kk_ref, vv_ref):
    t = tg_ref[...]
    kk_ref[...] = t @ wkg[...]
    vv_ref[...] = t @ wvg[...]


def _global_kernel(tg_ref, kk_ref, vv_ref, wqg, wog, wg1, bg1, wg2, bg2, z_ref):
    t = tg_ref[...]
    qq = t @ wqg[...]
    kk = kk_ref[...]
    vv = vv_ref[...]
    outs = []
    for h in range(H):
        s = lax.dot_general(qq[:, DH*h:DH*(h+1)], kk[:, DH*h:DH*(h+1)],
                            (((1,), (1,)), ((), ()))) * SCALE
        m = jnp.max(s, axis=1, keepdims=True)
        p = jnp.exp(s - m)
        a = p / jnp.sum(p, axis=1, keepdims=True)
        outs.append(a @ vv[:, DH*h:DH*(h+1)])
    z = t + jnp.concatenate(outs, axis=1) @ wog[...]
    z = z + jax.nn.gelu(z @ wg1[...] + bg1[...]) @ wg2[...] + bg2[...]
    z_ref[...] = z


def _full(shape):
    return pl.BlockSpec(shape, lambda i, *_: (0,) * len(shape))


def kernel(streams_data, segment_ids, pe_global, q_cells, W_embed, b_embed,
           Wq, Wk, Wv, Wo, W1, b1, W2, b2, Wq2, Wk2, Wv2, Wo2,
           Wqg, Wkg, Wvg, Wog, Wg1, bg1, Wg2, bg2):
    starts2, lens2 = pl.pallas_call(
        _seg_kernel,
        grid=(NSEG,),
        in_specs=[pl.BlockSpec((1, TB, 1), lambda i: (i, 0, 0))],
        out_specs=[pl.BlockSpec((1, C), lambda i: (0, 0))] * 2,
        out_shape=[jax.ShapeDtypeStruct((1, C), jnp.int32)] * 2,
        scratch_shapes=[pltpu.VMEM((1, C), jnp.float32)] * 2,
    )(segment_ids.reshape(NSEG, TB, 1))
    starts = starts2.reshape(C)
    lens = lens2.reshape(C)

    grid_spec = pltpu.PrefetchScalarGridSpec(
        num_scalar_prefetch=2,
        grid=(C // CB,),
        in_specs=[
            pl.BlockSpec(memory_space=pl.ANY),
            pl.BlockSpec((CB, Q, D), lambda i, *_: (i, 0, 0)),
            pl.BlockSpec((CB, Q, D), lambda i, *_: (i, 0, 0)),
            _full((SRC, D)), _full((1, D)),
            _full((D, D)), _full((D, D)), _full((D, D)), _full((D, D)),
            _full((D, FF)), _full((1, FF)), _full((FF, D)), _full((1, D)),
            _full((D, D)), _full((D, D)), _full((D, D)), _full((D, D)),
        ],
        out_specs=pl.BlockSpec((CB, Q, D), lambda i, *_: (i, 0, 0)),
        scratch_shapes=[pltpu.VMEM((CB, L, SRC), jnp.float32),
                        pltpu.SemaphoreType.DMA((CB,))],
    )
    tg = pl.pallas_call(
        _local_kernel,
        grid_spec=grid_spec,
        out_shape=jax.ShapeDtypeStruct((C, Q, D), jnp.float32),
    )(starts, lens, streams_data, q_cells, pe_global,
      W_embed, b_embed.reshape(1, D), Wq, Wk, Wv, Wo,
      W1, b1.reshape(1, FF), W2, b2.reshape(1, D), Wq2, Wk2, Wv2, Wo2)

    tg2 = tg.reshape(S, D)
    kk, vv = pl.pallas_call(
        _kv_kernel,
        grid=(S // QB,),
        in_specs=[pl.BlockSpec((QB, D), lambda i: (i, 0)),
                  pl.BlockSpec((D, D), lambda i: (0, 0)),
                  pl.BlockSpec((D, D), lambda i: (0, 0))],
        out_specs=[pl.BlockSpec((QB, D), lambda i: (i, 0))] * 2,
        out_shape=[jax.ShapeDtypeStruct((S, D), jnp.float32)] * 2,
    )(tg2, Wkg, Wvg)

    z = pl.pallas_call(
        _global_kernel,
        grid=(S // QB,),
        in_specs=[pl.BlockSpec((QB, D), lambda i: (i, 0)),
                  pl.BlockSpec((S, D), lambda i: (0, 0)),
                  pl.BlockSpec((S, D), lambda i: (0, 0)),
                  pl.BlockSpec((D, D), lambda i: (0, 0)),
                  pl.BlockSpec((D, D), lambda i: (0, 0)),
                  pl.BlockSpec((D, FF), lambda i: (0, 0)),
                  pl.BlockSpec((1, FF), lambda i: (0, 0)),
                  pl.BlockSpec((FF, D), lambda i: (0, 0)),
                  pl.BlockSpec((1, D), lambda i: (0, 0))],
        out_specs=pl.BlockSpec((QB, D), lambda i: (i, 0)),
        out_shape=jax.ShapeDtypeStruct((S, D), jnp.float32),
    )(tg2, kk, vv, Wqg, Wog, Wg1, bg1.reshape(1, FF), Wg2, bg2.reshape(1, D))
    return z
